# baseline (speedup 1.0000x reference)
import functools

import jax
import jax.numpy as jnp
from jax import lax
from jax.experimental import pallas as pl
from jax.experimental.pallas import tpu as pltpu
from jax.experimental.pallas import tpu_sc as plsc

NUM_ENTITIES = 1_000_000
DIM = 64
NI, NJ = 16384, 26
B = NI * NJ
NC, NS = 2, 16
NW = NC * NS
K = 128
IB = NI // K
M = (NJ * IB) // NW
BPW = M * K

_mesh = plsc.VectorSubcoreMesh(core_axis_name="c", subcore_axis_name="s")


@functools.partial(
    pl.kernel,
    mesh=_mesh,
    out_type=jax.ShapeDtypeStruct((NJ, DIM, NI), jnp.float32),
    compiler_params=pltpu.CompilerParams(needs_layout_passes=False),
    scratch_types=[
        pltpu.VMEM((BPW,), jnp.int32),
        pltpu.VMEM((BPW,), jnp.int32),
        pltpu.VMEM((4, K, 2 * DIM), jnp.float32),
        pltpu.VMEM((2, DIM, K), jnp.float32),
        pltpu.SemaphoreType.DMA,
        pltpu.SemaphoreType.DMA,
        pltpu.SemaphoreType.DMA,
        pltpu.SemaphoreType.DMA,
        pltpu.SemaphoreType.DMA,
        pltpu.SemaphoreType.DMA,
    ],
)
def _embed32(ids_hbm, w128_hbm, out_hbm, idx_v, ihi_v, g_v, t_v,
             gsem0, gsem1, gsem2, gsem3, ssem0, ssem1):
    wid = lax.axis_index("s") * NC + lax.axis_index("c")
    base = wid * BPW

    pltpu.sync_copy(ids_hbm.at[pl.ds(base, BPW)], idx_v)

    def _pre(k, _):
        sl = pl.ds(k * 16, 16)
        ihi_v[sl] = idx_v[sl] >> 1
        return _
    lax.fori_loop(0, BPW // 16, _pre, None)

    gsems = (gsem0, gsem1, gsem2, gsem3)
    ssems = (ssem0, ssem1)
    NG = len(gsems)

    def _fire_gather(m, s):
        pltpu.async_copy(
            w128_hbm.at[ihi_v.at[pl.ds(m * K, K)]], g_v.at[s], gsems[s])

    def _wait_gather(s):
        pltpu.make_async_copy(w128_hbm.at[ihi_v.at[pl.ds(0, K)]],
                              g_v.at[s], gsems[s]).wait()

    def _out_slice(m):
        gm = wid * M + m
        j = gm // IB
        i0 = (gm % IB) * K
        return out_hbm.at[j, :, pl.ds(i0, K)]

    def _fire_store(m, s):
        pltpu.async_copy(t_v.at[s], _out_slice(m), ssems[s])

    def _wait_store(s):
        pltpu.make_async_copy(t_v.at[s], _out_slice(0), ssems[s]).wait()

    for mm in range(NG):
        _fire_gather(mm, mm)

    iota16 = lax.iota(jnp.int32, 16)

    def _item(i, _):
        for sg in range(NG):
            m = NG * i + sg
            s = sg % 2
            _wait_gather(sg)

            @pl.when(m >= 2)
            def _():
                _wait_store(s)

            def _grp(g, _):
                sl = pl.ds(m * K + g * 16, 16)
                h = (idx_v[sl] & 1) << 6
                r = iota16 + g * 16
                col = pl.ds(g * 16, 16)

                def _loads(d0):
                    return [plsc.load_gather(g_v.at[sg], [r, h + (d0 + t)])
                            for t in range(16)]

                vals = _loads(0)
                for d0 in range(16, DIM, 16):
                    nxt = _loads(d0)
                    for t in range(16):
                        t_v[s, d0 - 16 + t, col] = vals[t]
                    vals = nxt
                for t in range(16):
                    t_v[s, DIM - 16 + t, col] = vals[t]
                return _
            lax.fori_loop(0, K // 16, _grp, None)

            _fire_store(m, s)

            @pl.when(m + NG < M)
            def _():
                _fire_gather(m + NG, sg)
        return _
    lax.fori_loop(0, M // NG, _item, None)

    _wait_store(0)
    _wait_store(1)


def kernel(ids, weight):
    ids_lin = jnp.transpose(ids).reshape(-1)
    w128 = weight.reshape(NUM_ENTITIES // 2, 2 * DIM)
    out_t = _embed32(ids_lin, w128)
    return jnp.transpose(out_t, (2, 0, 1))


# speedup vs baseline: 1.0249x; 1.0249x over previous
import functools

import jax
import jax.numpy as jnp
from jax import lax
from jax.experimental import pallas as pl
from jax.experimental.pallas import tpu as pltpu
from jax.experimental.pallas import tpu_sc as plsc

NUM_ENTITIES = 1_000_000
DIM = 64
B = 16384 * 26
NC, NS = 2, 16
NW = NC * NS
BPW = B // NW
CH = 832
NCHUNK = BPW // CH
NBUF = 2

_mesh = plsc.VectorSubcoreMesh(core_axis_name="c", subcore_axis_name="s")


@functools.partial(
    pl.kernel,
    mesh=_mesh,
    out_type=jax.ShapeDtypeStruct((B, DIM), jnp.float32),
    compiler_params=pltpu.CompilerParams(use_tc_tiling_on_sc=False),
    scratch_types=[
        pltpu.VMEM((BPW,), jnp.int32),
        pltpu.VMEM((NBUF, CH, DIM), jnp.float32),
        pltpu.SemaphoreType.DMA,
        pltpu.SemaphoreType.DMA,
    ],
)
def _gather32(ids_hbm, table_hbm, out_hbm, idx_v, rows_v, gsem, ssem):
    wid = lax.axis_index("s") * NC + lax.axis_index("c")
    base = wid * BPW
    pltpu.sync_copy(ids_hbm.at[pl.ds(base, BPW)], idx_v)

    gathers = [None] * NCHUNK
    stores = [None] * NCHUNK
    gathers[0] = pltpu.async_copy(
        table_hbm.at[idx_v.at[pl.ds(0, CH)]], rows_v.at[0], gsem)
    for c in range(NCHUNK):
        slot = c % NBUF
        gathers[c].wait()
        stores[c] = pltpu.async_copy(
            rows_v.at[slot], out_hbm.at[pl.ds(base + c * CH, CH)], ssem)
        if c + 1 < NCHUNK:
            if c - 1 >= 0:
                stores[c - 1].wait()
            gathers[c + 1] = pltpu.async_copy(
                table_hbm.at[idx_v.at[pl.ds((c + 1) * CH, CH)]],
                rows_v.at[(c + 1) % NBUF], gsem)
    if NCHUNK >= 2:
        stores[NCHUNK - 2].wait()
    stores[NCHUNK - 1].wait()


def kernel(ids, weight):
    ids_flat = ids.reshape(-1).astype(jnp.int32)
    out = _gather32(ids_flat, weight)
    return out.reshape(ids.shape + (weight.shape[1],))
